# baseline (speedup 1.0000x reference)
import functools
import math

import jax
import jax.numpy as jnp
from jax import lax
from jax.experimental import pallas as pl
from jax.experimental.pallas import tpu as pltpu
from jax.experimental.pallas import tpu_sc as plsc

D_MODEL = 64
SCALE = math.sqrt(D_MODEL)
LANES = 16
CHUNK = 128
NBUF = 4
SEQ = 4096
BATCH = 200


def _make_sc_kernel(b_per_w, n_chunks, num_cores):
    mesh = plsc.VectorSubcoreMesh(core_axis_name="c", subcore_axis_name="s")
    n_groups = n_chunks // NBUF

    @functools.partial(
        pl.kernel,
        mesh=mesh,
        compiler_params=pltpu.CompilerParams(use_tc_tiling_on_sc=False, needs_layout_passes=False),
        out_type=jax.ShapeDtypeStruct(
            (BATCH, 8, SEQ // CHUNK, 8, CHUNK), jnp.float32),
        scratch_types=[
            pltpu.VMEM((n_chunks, CHUNK), jnp.int32),
            *[pltpu.VMEM((CHUNK, D_MODEL), jnp.float32) for _ in range(NBUF)],
            *[pltpu.VMEM((8, 8, CHUNK + 1), jnp.float32) for _ in range(NBUF)],
            *[pltpu.SemaphoreType.DMA for _ in range(2 * NBUF)],
        ],
    )
    def sc_gather(table_hbm, idx_hbm, out_hbm, idx_all, *bufs_and_sems):
        rows = bufs_and_sems[:NBUF]
        tbuf = bufs_and_sems[NBUF:2 * NBUF]
        gsem = bufs_and_sems[2 * NBUF:3 * NBUF]
        osem = bufs_and_sems[3 * NBUF:]
        wid = lax.axis_index("s") * num_cores + lax.axis_index("c")
        base = wid * b_per_w

        pltpu.sync_copy(idx_hbm.at[wid], idx_all)
        for j in range(NBUF):
            pltpu.async_copy(table_hbm.at[idx_all.at[j]], rows[j], gsem[j])

        iota16 = lax.iota(jnp.int32, LANES)
        a_vecs = [(j0 * LANES + iota16) // 8 for j0 in range(D_MODEL // LANES)]
        c_vecs = [(j0 * LANES + iota16) % 8 for j0 in range(D_MODEL // LANES)]

        def group_body(g0, carry):
            for j in range(NBUF):
                g = g0 * NBUF + j
                b_glob = base + g * CHUNK
                s = b_glob // SEQ
                bp = (b_glob % SEQ) // CHUNK

                pltpu.make_async_copy(
                    table_hbm.at[idx_all.at[0]], rows[j], gsem[j]).wait()

                @pl.when(g0 > 0)
                def _wait_out():
                    pltpu.make_async_copy(
                        tbuf[j].at[:, :, pl.ds(0, CHUNK)],
                        out_hbm.at[0, :, 0], osem[j]).wait()

                rows_j = rows[j]
                tbuf_j = tbuf[j]

                @plsc.parallel_loop(0, CHUNK, step=1, unroll=8)
                def transpose_rows(d):
                    dsp = jnp.full((LANES,), d, jnp.int32)
                    for j0 in range(D_MODEL // LANES):
                        v = rows_j[d, pl.ds(j0 * LANES, LANES)]
                        plsc.store_scatter(
                            tbuf_j, [a_vecs[j0], c_vecs[j0], dsp],
                            v * SCALE)
                pltpu.async_copy(
                    tbuf[j].at[:, :, pl.ds(0, CHUNK)],
                    out_hbm.at[s, :, bp], osem[j])

                @pl.when(g + NBUF < n_chunks)
                def _issue_gather():
                    pltpu.async_copy(
                        table_hbm.at[idx_all.at[g + NBUF]], rows[j], gsem[j])
            return carry

        lax.fori_loop(0, n_groups, group_body, 0)
        for j in range(NBUF):
            pltpu.make_async_copy(
                tbuf[j].at[:, :, pl.ds(0, CHUNK)],
                out_hbm.at[0, :, 0], osem[j]).wait()

    return sc_gather


TCB = 512


def _tc_relayout_body(wt_ref, out_ref):
    eye = jnp.eye(D_MODEL, dtype=jnp.float32)
    out_ref[:, 0:D_MODEL] = lax.dot_general(
        wt_ref[...], eye, (((0,), (0,)), ((), ())),
        preferred_element_type=jnp.float32)
    out_ref[:, D_MODEL:] = jnp.zeros((TCB, 128 - D_MODEL), jnp.float32)


def _tc_relayout(wt):
    n = wt.shape[1]
    return pl.pallas_call(
        _tc_relayout_body,
        grid=(pl.cdiv(n, TCB),),
        in_specs=[pl.BlockSpec((D_MODEL, TCB), lambda i: (0, i))],
        out_specs=pl.BlockSpec((TCB, 128), lambda i: (i, 0)),
        out_shape=jax.ShapeDtypeStruct((n, 128), jnp.float32),
    )(wt)


def kernel(src, W):
    info = plsc.get_sparse_core_info()
    nw = info.num_cores * info.num_subcores
    idx = src.reshape(-1).astype(jnp.int32) * 2
    B = idx.shape[0]
    b_per_w = B // nw
    n_chunks = b_per_w // CHUNK
    idx3 = idx.reshape(nw, n_chunks, CHUNK)
    W2 = _tc_relayout(W.T).reshape(2 * W.shape[0], D_MODEL)
    sc_gather = _make_sc_kernel(b_per_w, n_chunks, info.num_cores)
    out5 = sc_gather(W2, idx3)
    out = out5.transpose(0, 2, 4, 1, 3).reshape(BATCH, SEQ, D_MODEL)
    return out


# speedup vs baseline: 2.0689x; 2.0689x over previous
import functools
import math

import jax
import jax.numpy as jnp
from jax import lax
from jax.experimental import pallas as pl
from jax.experimental.pallas import tpu as pltpu
from jax.experimental.pallas import tpu_sc as plsc

D_MODEL = 64
SCALE = math.sqrt(D_MODEL)
LANES = 16
CHUNK = 128
NBUF = 4
SEQ = 4096
BATCH = 200


def _make_sc_kernel(b_per_w, n_chunks, num_cores):
    mesh = plsc.VectorSubcoreMesh(core_axis_name="c", subcore_axis_name="s")
    n_groups = n_chunks // NBUF

    @functools.partial(
        pl.kernel,
        mesh=mesh,
        compiler_params=pltpu.CompilerParams(use_tc_tiling_on_sc=False, needs_layout_passes=False),
        out_type=jax.ShapeDtypeStruct(
            (BATCH, 8, SEQ // CHUNK, 8, CHUNK), jnp.float32),
        scratch_types=[
            pltpu.VMEM((n_chunks, CHUNK), jnp.int32),
            *[pltpu.VMEM((CHUNK, D_MODEL), jnp.float32) for _ in range(NBUF)],
            *[pltpu.VMEM((8, 8, CHUNK + 1), jnp.float32) for _ in range(NBUF)],
            *[pltpu.SemaphoreType.DMA for _ in range(2 * NBUF)],
        ],
    )
    def sc_gather(table_hbm, idx_hbm, out_hbm, idx_all, *bufs_and_sems):
        rows = bufs_and_sems[:NBUF]
        tbuf = bufs_and_sems[NBUF:2 * NBUF]
        gsem = bufs_and_sems[2 * NBUF:3 * NBUF]
        osem = bufs_and_sems[3 * NBUF:]
        wid = lax.axis_index("s") * num_cores + lax.axis_index("c")
        base = wid * b_per_w

        pltpu.sync_copy(idx_hbm.at[wid], idx_all)
        for j in range(NBUF):
            pltpu.async_copy(table_hbm.at[idx_all.at[j]], rows[j], gsem[j])

        iota16 = lax.iota(jnp.int32, LANES)
        a_vecs = [(j0 * LANES + iota16) // 8 for j0 in range(D_MODEL // LANES)]
        c_vecs = [(j0 * LANES + iota16) % 8 for j0 in range(D_MODEL // LANES)]

        def group_body(g0, carry):
            for j in range(NBUF):
                g = g0 * NBUF + j
                b_glob = base + g * CHUNK
                s = b_glob // SEQ
                bp = (b_glob % SEQ) // CHUNK

                pltpu.make_async_copy(
                    table_hbm.at[idx_all.at[0]], rows[j], gsem[j]).wait()

                @pl.when(g0 > 0)
                def _wait_out():
                    pltpu.make_async_copy(
                        tbuf[j].at[:, :, pl.ds(0, CHUNK)],
                        out_hbm.at[0, :, 0], osem[j]).wait()

                rows_j = rows[j]
                tbuf_j = tbuf[j]

                @plsc.parallel_loop(0, CHUNK, step=1, unroll=8)
                def transpose_rows(d):
                    dsp = jnp.full((LANES,), d, jnp.int32)
                    for j0 in range(D_MODEL // LANES):
                        v = rows_j[d, pl.ds(j0 * LANES, LANES)]
                        plsc.store_scatter(
                            tbuf_j, [a_vecs[j0], c_vecs[j0], dsp],
                            v * SCALE)
                pltpu.async_copy(
                    tbuf[j].at[:, :, pl.ds(0, CHUNK)],
                    out_hbm.at[s, :, bp], osem[j])

                @pl.when(g + NBUF < n_chunks)
                def _issue_gather():
                    pltpu.async_copy(
                        table_hbm.at[idx_all.at[g + NBUF]], rows[j], gsem[j])
            return carry

        lax.fori_loop(0, n_groups, group_body, 0)
        for j in range(NBUF):
            pltpu.make_async_copy(
                tbuf[j].at[:, :, pl.ds(0, CHUNK)],
                out_hbm.at[0, :, 0], osem[j]).wait()

    return sc_gather


def kernel(src, W):
    info = plsc.get_sparse_core_info()
    nw = info.num_cores * info.num_subcores
    idx = src.reshape(-1).astype(jnp.int32) * 2
    B = idx.shape[0]
    b_per_w = B // nw
    n_chunks = b_per_w // CHUNK
    idx3 = idx.reshape(nw, n_chunks, CHUNK)
    W2 = jnp.pad(W, ((0, 0), (0, 128 - D_MODEL))).reshape(
        2 * W.shape[0], D_MODEL)
    sc_gather = _make_sc_kernel(b_per_w, n_chunks, info.num_cores)
    out5 = sc_gather(W2, idx3)
    out = out5.transpose(0, 2, 4, 1, 3).reshape(BATCH, SEQ, D_MODEL)
    return out


# speedup vs baseline: 2.0784x; 1.0046x over previous
import functools
import math

import jax
import jax.numpy as jnp
from jax import lax
from jax.experimental import pallas as pl
from jax.experimental.pallas import tpu as pltpu
from jax.experimental.pallas import tpu_sc as plsc

D_MODEL = 64
SCALE = math.sqrt(D_MODEL)
LANES = 16
CHUNK = 128
NBUF = 4
SEQ = 4096
BATCH = 200


def _make_sc_kernel(b_per_w, n_chunks, num_cores):
    mesh = plsc.VectorSubcoreMesh(core_axis_name="c", subcore_axis_name="s")
    n_groups = n_chunks // NBUF

    @functools.partial(
        pl.kernel,
        mesh=mesh,
        compiler_params=pltpu.CompilerParams(use_tc_tiling_on_sc=False, needs_layout_passes=False),
        out_type=jax.ShapeDtypeStruct(
            (BATCH, 8, SEQ // CHUNK, 8, CHUNK), jnp.float32),
        scratch_types=[
            pltpu.VMEM((n_chunks, CHUNK), jnp.int32),
            *[pltpu.VMEM((CHUNK, D_MODEL), jnp.float32) for _ in range(NBUF)],
            *[pltpu.VMEM((8, 8, CHUNK + 1), jnp.float32) for _ in range(NBUF)],
            *[pltpu.SemaphoreType.DMA for _ in range(2 * NBUF)],
        ],
    )
    def sc_gather(table_hbm, idx_hbm, out_hbm, idx_all, *bufs_and_sems):
        rows = bufs_and_sems[:NBUF]
        tbuf = bufs_and_sems[NBUF:2 * NBUF]
        gsem = bufs_and_sems[2 * NBUF:3 * NBUF]
        osem = bufs_and_sems[3 * NBUF:]
        wid = lax.axis_index("s") * num_cores + lax.axis_index("c")
        base = wid * b_per_w

        pltpu.sync_copy(idx_hbm.at[wid], idx_all)
        for j in range(NBUF):
            pltpu.async_copy(table_hbm.at[idx_all.at[j]], rows[j], gsem[j])

        iota16 = lax.iota(jnp.int32, LANES)
        a_vecs = [(j0 * LANES + iota16) // 8 for j0 in range(D_MODEL // LANES)]
        c_vecs = [(j0 * LANES + iota16) % 8 for j0 in range(D_MODEL // LANES)]

        def group_body(g0, carry):
            for j in range(NBUF):
                g = g0 * NBUF + j
                b_glob = base + g * CHUNK
                s = b_glob // SEQ
                bp = (b_glob % SEQ) // CHUNK

                pltpu.make_async_copy(
                    table_hbm.at[idx_all.at[0]], rows[j], gsem[j]).wait()

                @pl.when(g0 > 0)
                def _wait_out():
                    pltpu.make_async_copy(
                        tbuf[j].at[:, :, pl.ds(0, CHUNK)],
                        out_hbm.at[0, :, 0], osem[j]).wait()

                rows_j = rows[j]
                tbuf_j = tbuf[j]

                @plsc.parallel_loop(0, CHUNK, step=1, unroll=16)
                def transpose_rows(d):
                    dsp = jnp.full((LANES,), d, jnp.int32)
                    for j0 in range(D_MODEL // LANES):
                        v = rows_j[d, pl.ds(j0 * LANES, LANES)]
                        plsc.store_scatter(
                            tbuf_j, [a_vecs[j0], c_vecs[j0], dsp],
                            v * SCALE)
                pltpu.async_copy(
                    tbuf[j].at[:, :, pl.ds(0, CHUNK)],
                    out_hbm.at[s, :, bp], osem[j])

                @pl.when(g + NBUF < n_chunks)
                def _issue_gather():
                    pltpu.async_copy(
                        table_hbm.at[idx_all.at[g + NBUF]], rows[j], gsem[j])
            return carry

        lax.fori_loop(0, n_groups, group_body, 0)
        for j in range(NBUF):
            pltpu.make_async_copy(
                tbuf[j].at[:, :, pl.ds(0, CHUNK)],
                out_hbm.at[0, :, 0], osem[j]).wait()

    return sc_gather


def kernel(src, W):
    info = plsc.get_sparse_core_info()
    nw = info.num_cores * info.num_subcores
    idx = src.reshape(-1).astype(jnp.int32) * 2
    B = idx.shape[0]
    b_per_w = B // nw
    n_chunks = b_per_w // CHUNK
    idx3 = idx.reshape(nw, n_chunks, CHUNK)
    W2 = jnp.pad(W, ((0, 0), (0, 128 - D_MODEL))).reshape(
        2 * W.shape[0], D_MODEL)
    sc_gather = _make_sc_kernel(b_per_w, n_chunks, info.num_cores)
    out5 = sc_gather(W2, idx3)
    out = out5.transpose(0, 2, 4, 1, 3).reshape(BATCH, SEQ, D_MODEL)
    return out
